# Initial kernel scaffold; baseline (speedup 1.0000x reference)
import jax
import jax.numpy as jnp
from jax.experimental import pallas as pl


def kernel(x, W_in, W_out):
    raise NotImplementedError("write your pallas kernel here")


# speedup vs baseline: 4.9864x; 4.9864x over previous
import jax
import jax.numpy as jnp
from jax import lax
from jax.experimental import pallas as pl
from jax.experimental.pallas import tpu as pltpu

_DEPTH = 11
_STEPS = _DEPTH + 1
_N_NODES = 2 ** (_DEPTH + 1) - 1
_N_PAD = 2 ** (_DEPTH + 1)
_BB = 512


def _seg_bounds(d):
    base = (1 << d) - 1
    width = 1 << d
    lo = (base // 128) * 128
    hi = min(-(-(base + width) // 128) * 128, _N_PAD)
    return base, lo, hi


def _route_kernel(x_ref, win_ref, nodes_ref, g_ref, logits_ref):
    logits_ref[...] = lax.dot_general(
        x_ref[...], win_ref[...],
        dimension_numbers=(((1,), (1,)), ((), ())),
        preferred_element_type=jnp.float32,
    )
    bb = x_ref.shape[0]
    l = jnp.zeros((bb, 1), jnp.int32)
    for d in range(_STEPS):
        base, lo, hi = _seg_bounds(d)
        seg = logits_ref[:, lo:hi]
        col = lax.broadcasted_iota(jnp.int32, (bb, hi - lo), 1) + lo
        node = base + l
        score = jnp.sum(jnp.where(col == node, seg, 0.0), axis=1, keepdims=True)
        nodes_ref[:, d:d + 1] = node
        g_ref[:, d:d + 1] = score * 0.5 * (1.0 + lax.erf(score * (2.0 ** -0.5)))
        l = 2 * l + (score >= 0).astype(jnp.int32)


def _out_kernel(nodes_ref, g_ref, wout_ref, o_ref, coeff_ref):
    bb = o_ref.shape[0]
    coeff_ref[...] = jnp.zeros_like(coeff_ref)
    for d in range(_STEPS):
        _, lo, hi = _seg_bounds(d)
        col = lax.broadcasted_iota(jnp.int32, (bb, hi - lo), 1) + lo
        node = nodes_ref[:, d:d + 1]
        g = g_ref[:, d:d + 1]
        coeff_ref[:, lo:hi] += jnp.where(col == node, g, 0.0)
    o_ref[...] = lax.dot_general(
        coeff_ref[...], wout_ref[...],
        dimension_numbers=(((1,), (1,)), ((), ())),
        preferred_element_type=jnp.float32,
    )


def kernel(x, W_in, W_out):
    B, D_in = x.shape
    D_out = W_out.shape[0]
    pad = _N_PAD - _N_NODES
    win = jnp.pad(W_in, ((0, pad), (0, 0)))
    wout = jnp.pad(W_out, ((0, 0), (0, pad)))
    grid = (B // _BB,)

    nodes, g = pl.pallas_call(
        _route_kernel,
        grid=grid,
        in_specs=[
            pl.BlockSpec((_BB, D_in), lambda i: (i, 0)),
            pl.BlockSpec((_N_PAD, D_in), lambda i: (0, 0)),
        ],
        out_specs=[
            pl.BlockSpec((_BB, _STEPS), lambda i: (i, 0)),
            pl.BlockSpec((_BB, _STEPS), lambda i: (i, 0)),
        ],
        out_shape=[
            jax.ShapeDtypeStruct((B, _STEPS), jnp.int32),
            jax.ShapeDtypeStruct((B, _STEPS), jnp.float32),
        ],
        scratch_shapes=[pltpu.VMEM((_BB, _N_PAD), jnp.float32)],
        compiler_params=pltpu.CompilerParams(
            dimension_semantics=("parallel",)),
    )(x, win)

    out = pl.pallas_call(
        _out_kernel,
        grid=grid,
        in_specs=[
            pl.BlockSpec((_BB, _STEPS), lambda i: (i, 0)),
            pl.BlockSpec((_BB, _STEPS), lambda i: (i, 0)),
            pl.BlockSpec((D_out, _N_PAD), lambda i: (0, 0)),
        ],
        out_specs=pl.BlockSpec((_BB, D_out), lambda i: (i, 0)),
        out_shape=jax.ShapeDtypeStruct((B, D_out), jnp.float32),
        scratch_shapes=[pltpu.VMEM((_BB, _N_PAD), jnp.float32)],
        compiler_params=pltpu.CompilerParams(
            dimension_semantics=("parallel",)),
    )(nodes, g, wout)
    return out
